# baseline (speedup 1.0000x reference)
import jax
import jax.numpy as jnp
from jax.experimental import pallas as pl
from jax.experimental.pallas import tpu as pltpu

_V = 200
_BLOCK_B = 64
_SUB_B = 8


def _fused_kernel(idx_ref, tgt_ref, emb_ref, logits_ref, losssum_ref, *, sub):
    emb = emb_ref[...]
    rb, T = idx_ref.shape
    v_pad = emb.shape[1]
    n = sub * T

    emb_hi = emb.astype(jnp.bfloat16)
    emb_lo = (emb - emb_hi.astype(jnp.float32)).astype(jnp.bfloat16)

    m = jnp.max(emb, axis=1, keepdims=True)
    lse_vec = m + jnp.log(jnp.sum(jnp.exp(emb - m), axis=1, keepdims=True))
    lse_hi = lse_vec.astype(jnp.bfloat16)
    lse_lo = (lse_vec - lse_hi.astype(jnp.float32)).astype(jnp.bfloat16)

    row = jax.lax.broadcasted_iota(jnp.int32, (n, 1), 0)
    colb = jax.lax.broadcasted_iota(jnp.int32, (n, sub), 1)
    sel = ((row // T) == colb).astype(jnp.bfloat16)
    colt = jax.lax.broadcasted_iota(jnp.int32, (n, T), 1)
    tmask = colt == (row % T)
    colv = jax.lax.broadcasted_iota(jnp.int32, (n, v_pad), 1)

    acc = jnp.zeros((1, 1), jnp.float32)
    for j in range(rb // sub):
        idx_blk = idx_ref[pl.ds(j * sub, sub), :].astype(jnp.bfloat16)
        tgt_blk = tgt_ref[pl.ds(j * sub, sub), :].astype(jnp.bfloat16)

        rows_idx = jnp.dot(sel, idx_blk, preferred_element_type=jnp.float32)
        idx_flat = jnp.sum(jnp.where(tmask, rows_idx, 0.0),
                           axis=1, keepdims=True)
        rows_tgt = jnp.dot(sel, tgt_blk, preferred_element_type=jnp.float32)
        tgt_flat = jnp.sum(jnp.where(tmask, rows_tgt, 0.0),
                           axis=1, keepdims=True)

        onehot = (colv == idx_flat.astype(jnp.int32)).astype(jnp.bfloat16)
        logits = (jnp.dot(onehot, emb_hi, preferred_element_type=jnp.float32)
                  + jnp.dot(onehot, emb_lo, preferred_element_type=jnp.float32))
        logits_ref[pl.ds(j * n, n), :] = logits

        row_lse = (jnp.dot(onehot, lse_hi, preferred_element_type=jnp.float32)
                   + jnp.dot(onehot, lse_lo, preferred_element_type=jnp.float32))
        tgt_logit = jnp.sum(jnp.where(colv == tgt_flat.astype(jnp.int32),
                                      logits, 0.0), axis=1, keepdims=True)
        acc = acc + jnp.sum(row_lse - tgt_logit, keepdims=True)
    losssum_ref[...] = acc[None]


@jax.jit
def kernel(idx, targets, emb_padded):
    B, T = idx.shape
    V_pad = emb_padded.shape[1]
    N = B * T
    tile_n = _BLOCK_B * T
    num_tiles = B // _BLOCK_B

    cost = pl.CostEstimate(
        flops=2 * N * V_pad * V_pad,
        transcendentals=num_tiles * V_pad * V_pad,
        bytes_accessed=2 * N * 4 + V_pad * V_pad * 4 + N * _V * 4)
    import functools as _ft
    logits, loss_sums = pl.pallas_call(
        _ft.partial(_fused_kernel, sub=_SUB_B),
        out_shape=(
            jax.ShapeDtypeStruct((N, V_pad), jnp.float32),
            jax.ShapeDtypeStruct((num_tiles, 1, 1), jnp.float32),
        ),
        grid=(num_tiles,),
        in_specs=[
            pl.BlockSpec((_BLOCK_B, T), lambda i: (i, 0)),
            pl.BlockSpec((_BLOCK_B, T), lambda i: (i, 0)),
            pl.BlockSpec((V_pad, V_pad), lambda i: (0, 0)),
        ],
        out_specs=(
            pl.BlockSpec((tile_n, V_pad), lambda i: (i, 0)),
            pl.BlockSpec((1, 1, 1), lambda i: (i, 0, 0)),
        ),
        compiler_params=pltpu.CompilerParams(
            dimension_semantics=("parallel",),
            vmem_limit_bytes=64 * 1024 * 1024,
        ),
        cost_estimate=cost,
    )(idx, targets, emb_padded)

    loss = jnp.sum(loss_sums) / jnp.float32(N)
    return logits[:, :_V], loss


# speedup vs baseline: 1.2092x; 1.2092x over previous
import jax
import jax.numpy as jnp
from jax.experimental import pallas as pl
from jax.experimental.pallas import tpu as pltpu

_V = 200
_BLOCK_B = 64
_SUB_B = 8
_SLICE_CHUNKS = 4


def _fused_kernel(idx_ref, tgt_ref, emb_ref, logits_ref, losssum_ref, *, sub):
    emb = emb_ref[...]
    rb, T = idx_ref.shape
    v_pad = emb.shape[1]
    n = sub * T

    m = jnp.max(emb, axis=1, keepdims=True)
    lse_vec = m + jnp.log(jnp.sum(jnp.exp(emb - m), axis=1, keepdims=True))

    row = jax.lax.broadcasted_iota(jnp.int32, (n, 1), 0)
    colb = jax.lax.broadcasted_iota(jnp.int32, (n, sub), 1)
    sel = ((row // T) == colb).astype(jnp.float32)
    colt = jax.lax.broadcasted_iota(jnp.int32, (n, T), 1)
    tmask = colt == (row % T)
    colv = jax.lax.broadcasted_iota(jnp.int32, (n, v_pad), 1)

    acc = jnp.zeros((1, 1), jnp.float32)
    for j in range(rb // sub):
        idx_blk = idx_ref[pl.ds(j * sub, sub), :].astype(jnp.float32)
        tgt_blk = tgt_ref[pl.ds(j * sub, sub), :].astype(jnp.float32)

        rows_idx = jnp.dot(sel, idx_blk, preferred_element_type=jnp.float32)
        idx_flat = jnp.sum(jnp.where(tmask, rows_idx, 0.0),
                           axis=1, keepdims=True)
        rows_tgt = jnp.dot(sel, tgt_blk, preferred_element_type=jnp.float32)
        tgt_flat = jnp.sum(jnp.where(tmask, rows_tgt, 0.0),
                           axis=1, keepdims=True)

        onehot = (colv == idx_flat.astype(jnp.int32)).astype(jnp.float32)
        logits = jnp.dot(onehot, emb, preferred_element_type=jnp.float32)
        logits_ref[pl.ds(j * n, n), :] = logits

        row_lse = jnp.dot(onehot, lse_vec, preferred_element_type=jnp.float32)
        tgt_logit = jnp.sum(jnp.where(colv == tgt_flat.astype(jnp.int32),
                                      logits, 0.0), axis=1, keepdims=True)
        acc = acc + jnp.sum(row_lse - tgt_logit, keepdims=True)
    losssum_ref[...] = acc[None]


@jax.jit
def kernel(idx, targets, emb_padded):
    B, T = idx.shape
    V_pad = emb_padded.shape[1]
    N = B * T
    tile_n = _BLOCK_B * T
    num_tiles = B // _BLOCK_B

    cost = pl.CostEstimate(
        flops=2 * N * V_pad * V_pad,
        transcendentals=num_tiles * V_pad * V_pad,
        bytes_accessed=2 * N * 4 + V_pad * V_pad * 4 + N * _V * 4)
    import functools as _ft
    logits, loss_sums = pl.pallas_call(
        _ft.partial(_fused_kernel, sub=_SUB_B),
        out_shape=(
            jax.ShapeDtypeStruct((N, V_pad), jnp.float32),
            jax.ShapeDtypeStruct((num_tiles, 1, 1), jnp.float32),
        ),
        grid=(num_tiles,),
        in_specs=[
            pl.BlockSpec((_BLOCK_B, T), lambda i: (i, 0)),
            pl.BlockSpec((_BLOCK_B, T), lambda i: (i, 0)),
            pl.BlockSpec((V_pad, V_pad), lambda i: (0, 0)),
        ],
        out_specs=(
            pl.BlockSpec((tile_n, V_pad), lambda i: (i, 0)),
            pl.BlockSpec((1, 1, 1), lambda i: (i, 0, 0)),
        ),
        compiler_params=pltpu.CompilerParams(
            dimension_semantics=("arbitrary",),
            vmem_limit_bytes=64 * 1024 * 1024,
        ),
        cost_estimate=cost,
    )(idx, targets, emb_padded)

    loss = jnp.sum(loss_sums) / jnp.float32(N)
    nc = N // _SLICE_CHUNKS
    parts = [jax.lax.slice(logits, (c * nc, 0), ((c + 1) * nc, _V))
             for c in range(_SLICE_CHUNKS)]
    return jnp.concatenate(parts, axis=0), loss


# speedup vs baseline: 1.2282x; 1.0157x over previous
import jax
import jax.numpy as jnp
from jax.experimental import pallas as pl
from jax.experimental.pallas import tpu as pltpu

_V = 200
_BLOCK_B = 32


def _fused_kernel(idx_ref, tgt_ref, emb_ref, logits_ref, losssum_ref):
    idx_blk = idx_ref[...].astype(jnp.float32)
    tgt_blk = tgt_ref[...].astype(jnp.float32)
    emb = emb_ref[...]
    rb, T = idx_blk.shape
    n = rb * T
    v_pad = emb.shape[1]

    row = jax.lax.broadcasted_iota(jnp.int32, (n, 1), 0)
    colb = jax.lax.broadcasted_iota(jnp.int32, (n, rb), 1)
    sel = ((row // T) == colb).astype(jnp.float32)
    colt = jax.lax.broadcasted_iota(jnp.int32, (n, T), 1)
    tmask = colt == (row % T)

    rows_idx = jnp.dot(sel, idx_blk, preferred_element_type=jnp.float32)
    idx_flat = jnp.sum(jnp.where(tmask, rows_idx, 0.0), axis=1, keepdims=True)
    rows_tgt = jnp.dot(sel, tgt_blk, preferred_element_type=jnp.float32)
    tgt_flat = jnp.sum(jnp.where(tmask, rows_tgt, 0.0), axis=1, keepdims=True)

    colv = jax.lax.broadcasted_iota(jnp.int32, (n, v_pad), 1)
    onehot = (colv == idx_flat.astype(jnp.int32)).astype(jnp.float32)
    logits = jnp.dot(onehot, emb, preferred_element_type=jnp.float32)
    logits_ref[...] = logits

    m = jnp.max(emb, axis=1, keepdims=True)
    lse_vec = m + jnp.log(jnp.sum(jnp.exp(emb - m), axis=1, keepdims=True))
    row_lse = jnp.dot(onehot, lse_vec, preferred_element_type=jnp.float32)

    tgt_logit = jnp.sum(jnp.where(colv == tgt_flat.astype(jnp.int32),
                                  logits, 0.0), axis=1, keepdims=True)
    losssum_ref[...] = jnp.sum(row_lse - tgt_logit, keepdims=True)[None]


@jax.jit
def kernel(idx, targets, emb_padded):
    B, T = idx.shape
    V_pad = emb_padded.shape[1]
    N = B * T
    tile_n = _BLOCK_B * T
    num_tiles = B // _BLOCK_B

    cost = pl.CostEstimate(
        flops=2 * N * V_pad * V_pad,
        transcendentals=num_tiles * V_pad * V_pad,
        bytes_accessed=2 * N * 4 + V_pad * V_pad * 4 + N * V_pad * 4)
    logits, loss_sums = pl.pallas_call(
        _fused_kernel,
        out_shape=(
            jax.ShapeDtypeStruct((N, V_pad), jnp.float32),
            jax.ShapeDtypeStruct((num_tiles, 1, 1), jnp.float32),
        ),
        grid=(num_tiles,),
        in_specs=[
            pl.BlockSpec((_BLOCK_B, T), lambda i: (i, 0)),
            pl.BlockSpec((_BLOCK_B, T), lambda i: (i, 0)),
            pl.BlockSpec((V_pad, V_pad), lambda i: (0, 0)),
        ],
        out_specs=(
            pl.BlockSpec((tile_n, V_pad), lambda i: (i, 0)),
            pl.BlockSpec((1, 1, 1), lambda i: (i, 0, 0)),
        ),
        compiler_params=pltpu.CompilerParams(
            dimension_semantics=("parallel",),
            vmem_limit_bytes=64 * 1024 * 1024,
        ),
        cost_estimate=cost,
    )(idx, targets, emb_padded)

    loss = jnp.sum(loss_sums) / jnp.float32(N)
    return logits[:, :_V], loss


# speedup vs baseline: 1.3214x; 1.0758x over previous
import jax
import jax.numpy as jnp
from jax.experimental import pallas as pl
from jax.experimental.pallas import tpu as pltpu

_V = 200
_BLOCK_B = 32


def _fused_kernel(idx_ref, tgt_ref, emb_ref, logits_ref, losssum_ref):
    emb = emb_ref[...]
    rb, T = idx_ref.shape
    n = rb * T
    v_pad = emb.shape[1]

    packed_blk = (idx_ref[...] + 1024 * tgt_ref[...]).astype(jnp.float32)

    row = jax.lax.broadcasted_iota(jnp.int32, (n, 1), 0)
    colb = jax.lax.broadcasted_iota(jnp.int32, (n, rb), 1)
    sel = ((row // T) == colb).astype(jnp.float32)
    colt = jax.lax.broadcasted_iota(jnp.int32, (n, T), 1)
    tmask = colt == (row % T)

    rows_p = jnp.dot(sel, packed_blk, preferred_element_type=jnp.float32)
    val = jnp.sum(jnp.where(tmask, rows_p, 0.0), axis=1, keepdims=True)
    val_i = val.astype(jnp.int32)
    tgt_i = jax.lax.shift_right_logical(val_i, 10)
    idx_i = jax.lax.bitwise_and(val_i, 1023)

    colv = jax.lax.broadcasted_iota(jnp.int32, (n, v_pad), 1)
    onehot = (colv == idx_i).astype(jnp.float32)
    logits = jnp.dot(onehot, emb, preferred_element_type=jnp.float32)
    logits_ref[...] = logits

    m = jnp.max(emb, axis=1, keepdims=True)
    lse_vec = m + jnp.log(jnp.sum(jnp.exp(emb - m), axis=1, keepdims=True))
    counts = jnp.dot(jnp.ones((8, n), jnp.float32), onehot,
                     preferred_element_type=jnp.float32)
    lse_sum = jnp.dot(counts[0:1, :], lse_vec,
                      preferred_element_type=jnp.float32)

    tgt_sum = jnp.sum(jnp.where(colv == tgt_i, logits, 0.0), keepdims=True)
    losssum_ref[...] = (lse_sum - tgt_sum[0:1, 0:1])[None]


@jax.jit
def kernel(idx, targets, emb_padded):
    B, T = idx.shape
    V_pad = emb_padded.shape[1]
    N = B * T
    tile_n = _BLOCK_B * T
    num_tiles = B // _BLOCK_B

    cost = pl.CostEstimate(
        flops=2 * N * V_pad * V_pad,
        transcendentals=num_tiles * V_pad * V_pad,
        bytes_accessed=2 * N * 4 + V_pad * V_pad * 4 + N * V_pad * 4)
    logits, loss_sums = pl.pallas_call(
        _fused_kernel,
        out_shape=(
            jax.ShapeDtypeStruct((N, V_pad), jnp.float32),
            jax.ShapeDtypeStruct((num_tiles, 1, 1), jnp.float32),
        ),
        grid=(num_tiles,),
        in_specs=[
            pl.BlockSpec((_BLOCK_B, T), lambda i: (i, 0)),
            pl.BlockSpec((_BLOCK_B, T), lambda i: (i, 0)),
            pl.BlockSpec((V_pad, V_pad), lambda i: (0, 0)),
        ],
        out_specs=(
            pl.BlockSpec((tile_n, V_pad), lambda i: (i, 0)),
            pl.BlockSpec((1, 1, 1), lambda i: (i, 0, 0)),
        ),
        compiler_params=pltpu.CompilerParams(
            dimension_semantics=("parallel",),
            vmem_limit_bytes=64 * 1024 * 1024,
        ),
        cost_estimate=cost,
    )(idx, targets, emb_padded)

    loss = jnp.sum(loss_sums) / jnp.float32(N)
    return logits[:, :_V], loss
